# Initial kernel scaffold; baseline (speedup 1.0000x reference)
import jax
import jax.numpy as jnp
from jax import lax
from jax.experimental import pallas as pl
from jax.experimental.pallas import tpu as pltpu
from jax.experimental.pallas import tpu_sc as plsc

NC, NS, L = 2, 16, 16
NW = NC * NS
B, C, H, W = 32, 3, 512, 512
GH, GW = 8, 8
TH, TW = H // GH, W // GW
NBINS = 256
NPIX = TH * TW
NIMG = B * C
IPW = NIMG // NW
CLIP_VAL = 64.0
LUT_PER_IMG = GH * GW * NBINS


def _hist_body(x_hbm, luts_hbm, x_v, sub_v, hist_v, lut_v):
    wid = lax.axis_index("s") * NC + lax.axis_index("c")
    laneoff = lax.iota(jnp.int32, L) * NBINS
    ones = jnp.ones((L,), jnp.float32)
    zeros = jnp.zeros((L,), jnp.float32)

    for il in range(IPW):
        img = wid * IPW + il
        bi = img // C
        ci = img % C

        def band_body(band, _):
            pltpu.sync_copy(x_hbm.at[bi, ci, pl.ds(band * TH, TH), :], x_v)

            def tile_body(tc, _):
                def z_body(i, _):
                    sub_v[pl.ds(i * L, L)] = zeros
                    return 0
                lax.fori_loop(0, (NBINS * L) // L, z_body, 0)

                def px_body(i, _):
                    r = i // (TW // L)
                    k = i % (TW // L)
                    xv = x_v[r, pl.ds(tc * TW + k * L, L)]
                    t = xv / 255.0 * 256.0
                    bn = jnp.clip(t.astype(jnp.int32), 0, NBINS - 1)
                    plsc.addupdate_scatter(sub_v, [bn + laneoff], ones)
                    return 0
                lax.fori_loop(0, NPIX // L, px_body, 0)

                def red_body(cc, _):
                    acc = sub_v[pl.ds(cc * L, L)]
                    for l in range(1, L):
                        acc = acc + sub_v[pl.ds(l * NBINS + cc * L, L)]
                    hist_v[pl.ds(cc * L, L)] = acc
                    return 0
                lax.fori_loop(0, NBINS // L, red_body, 0)

                def clip_body(cc, s):
                    h = hist_v[pl.ds(cc * L, L)]
                    cl = jnp.minimum(h, CLIP_VAL)
                    hist_v[pl.ds(cc * L, L)] = cl
                    return s + jnp.sum(cl)
                sumc = lax.fori_loop(0, NBINS // L, clip_body, jnp.float32(0.0))
                exc = (jnp.float32(NPIX) - sumc) * (1.0 / NBINS)

                def lut_body(cc, carry):
                    h = hist_v[pl.ds(cc * L, L)] + exc
                    cs = plsc.cumsum(h) + carry
                    lut = (cs * 255.0 / float(NPIX)).astype(jnp.int32)
                    lutf = jnp.clip(lut.astype(jnp.float32), 0.0, 255.0)
                    lut_v[pl.ds(tc * NBINS + cc * L, L)] = lutf
                    return carry + jnp.sum(h)
                lax.fori_loop(0, NBINS // L, lut_body, jnp.float32(0.0))
                return 0

            lax.fori_loop(0, GW, tile_body, 0)
            pltpu.sync_copy(
                lut_v, luts_hbm.at[img, pl.ds(band * GW * NBINS, GW * NBINS)])
            return 0

        lax.fori_loop(0, GH, band_body, 0)


def _apply_body(x_hbm, luts_hbm, out_hbm, x_v, o_v, lut_v, co0_v, co1_v, wx_v):
    wid = lax.axis_index("s") * NC + lax.axis_index("c")
    lanes = lax.iota(jnp.int32, L)

    def col_body(k, _):
        xpos = (k * L + lanes).astype(jnp.float32)
        tx = (xpos + 0.5) / float(TW) - 0.5
        x0i = (tx + 1.0).astype(jnp.int32) - 1
        x0 = jnp.clip(x0i, 0, GW - 1)
        x1 = jnp.clip(x0i + 1, 0, GW - 1)
        co0_v[pl.ds(k * L, L)] = x0 * NBINS
        co1_v[pl.ds(k * L, L)] = x1 * NBINS
        wx_v[pl.ds(k * L, L)] = tx - x0i.astype(jnp.float32)
        return 0
    lax.fori_loop(0, W // L, col_body, 0)

    for il in range(IPW):
        img = wid * IPW + il
        bi = img // C
        ci = img % C
        pltpu.sync_copy(luts_hbm.at[img], lut_v)

        def band_body(band, _):
            pltpu.sync_copy(x_hbm.at[bi, ci, pl.ds(band * TH, TH), :], x_v)

            def row_body(r, _):
                y = band * TH + r
                ty = (y.astype(jnp.float32) + 0.5) / float(TH) - 0.5
                y0i = (ty + 1.0).astype(jnp.int32) - 1
                y0 = jnp.clip(y0i, 0, GH - 1)
                y1 = jnp.clip(y0i + 1, 0, GH - 1)
                wy = ty - y0i.astype(jnp.float32)
                ro0 = y0 * (GW * NBINS)
                ro1 = y1 * (GW * NBINS)

                def ch_body(k, _):
                    xv = x_v[r, pl.ds(k * L, L)]
                    t = xv / 255.0 * 256.0
                    bn = jnp.clip(t.astype(jnp.int32), 0, NBINS - 1)
                    s0 = bn + co0_v[pl.ds(k * L, L)]
                    s1 = bn + co1_v[pl.ds(k * L, L)]
                    wx = wx_v[pl.ds(k * L, L)]
                    v00 = plsc.load_gather(lut_v, [s0 + ro0])
                    v01 = plsc.load_gather(lut_v, [s1 + ro0])
                    v10 = plsc.load_gather(lut_v, [s0 + ro1])
                    v11 = plsc.load_gather(lut_v, [s1 + ro1])
                    omx = 1.0 - wx
                    interp = ((1.0 - wy) * (omx * v00 + wx * v01)
                              + wy * (omx * v10 + wx * v11))
                    o_v[r, pl.ds(k * L, L)] = interp / 255.0 * 255.0
                    return 0
                lax.fori_loop(0, W // L, ch_body, 0)
                return 0

            lax.fori_loop(0, TH, row_body, 0)
            pltpu.sync_copy(o_v, out_hbm.at[bi, ci, pl.ds(band * TH, TH), :])
            return 0

        lax.fori_loop(0, GH, band_body, 0)


_mesh = plsc.VectorSubcoreMesh(core_axis_name="c", subcore_axis_name="s",
                               num_cores=NC, num_subcores=NS)

_hist_call = pl.kernel(
    _hist_body,
    out_type=jax.ShapeDtypeStruct((NIMG, LUT_PER_IMG), jnp.float32),
    mesh=_mesh,
    scratch_types=[
        pltpu.VMEM((TH, W), jnp.float32),
        pltpu.VMEM((NBINS * L,), jnp.float32),
        pltpu.VMEM((NBINS,), jnp.float32),
        pltpu.VMEM((GW * NBINS,), jnp.float32),
    ],
)

_apply_call = pl.kernel(
    _apply_body,
    out_type=jax.ShapeDtypeStruct((B, C, H, W), jnp.float32),
    mesh=_mesh,
    scratch_types=[
        pltpu.VMEM((TH, W), jnp.float32),
        pltpu.VMEM((TH, W), jnp.float32),
        pltpu.VMEM((LUT_PER_IMG,), jnp.float32),
        pltpu.VMEM((W,), jnp.int32),
        pltpu.VMEM((W,), jnp.int32),
        pltpu.VMEM((W,), jnp.float32),
    ],
)


@jax.jit
def kernel(x):
    luts = _hist_call(x)
    return _apply_call(x, luts)


# speedup vs baseline: 1123.5549x; 1123.5549x over previous
import jax
import jax.numpy as jnp
from jax import lax
from jax.experimental import pallas as pl
from jax.experimental.pallas import tpu as pltpu
from jax.experimental.pallas import tpu_sc as plsc

NC, NS, L = 2, 16, 16
NW = NC * NS
B, C, H, W = 32, 3, 512, 512
GH, GW = 8, 8
TH, TW = H // GH, W // GW
NBINS = 256
NPIX = TH * TW
NIMG = B * C
IPW = NIMG // NW
CLIP_VAL = 64.0
LUT_PER_IMG = GH * GW * NBINS


def _hist_body(x_hbm, luts_hbm, x_v, sub_v, hist_v, lut_v):
    wid = lax.axis_index("s") * NC + lax.axis_index("c")
    laneoff = lax.iota(jnp.int32, L) * NBINS
    ones = jnp.ones((L,), jnp.float32)
    zeros = jnp.zeros((L,), jnp.float32)

    for il in range(IPW):
        img = wid * IPW + il
        bi = img // C
        ci = img % C

        def band_body(band, _):
            pltpu.sync_copy(x_hbm.at[bi, ci, pl.ds(band * TH, TH), :], x_v)

            def tile_body(tc, _):
                def z_body(i, _):
                    sub_v[pl.ds(i * L, L)] = zeros
                    return 0
                lax.fori_loop(0, (NBINS * L) // L, z_body, 0)

                def px_body(i, _):
                    r = i // (TW // L)
                    k = i % (TW // L)
                    xv = x_v[r, pl.ds(tc * TW + k * L, L)]
                    t = xv * (1.0 / 255.0) * 256.0
                    bn = jnp.clip(t.astype(jnp.int32), 0, NBINS - 1)
                    plsc.addupdate_scatter(sub_v, [bn + laneoff], ones)
                    return 0
                lax.fori_loop(0, NPIX // L, px_body, 0)

                def red_body(cc, _):
                    acc = sub_v[pl.ds(cc * L, L)]
                    for l in range(1, L):
                        acc = acc + sub_v[pl.ds(l * NBINS + cc * L, L)]
                    hist_v[pl.ds(cc * L, L)] = acc
                    return 0
                lax.fori_loop(0, NBINS // L, red_body, 0)

                def clip_body(cc, s):
                    h = hist_v[pl.ds(cc * L, L)]
                    cl = jnp.minimum(h, CLIP_VAL)
                    hist_v[pl.ds(cc * L, L)] = cl
                    return s + jnp.sum(cl)
                sumc = lax.fori_loop(0, NBINS // L, clip_body, jnp.float32(0.0))
                exc = (jnp.float32(NPIX) - sumc) * (1.0 / NBINS)

                def lut_body(cc, carry):
                    h = hist_v[pl.ds(cc * L, L)] + exc
                    cs = plsc.cumsum(h) + carry
                    lut = (cs * 255.0 * (1.0 / NPIX)).astype(jnp.int32)
                    lutf = jnp.clip(lut.astype(jnp.float32), 0.0, 255.0)
                    lut_v[pl.ds(tc * NBINS + cc * L, L)] = lutf
                    return carry + jnp.sum(h)
                lax.fori_loop(0, NBINS // L, lut_body, jnp.float32(0.0))
                return 0

            lax.fori_loop(0, GW, tile_body, 0)
            pltpu.sync_copy(
                lut_v, luts_hbm.at[img, pl.ds(band * GW * NBINS, GW * NBINS)])
            return 0

        lax.fori_loop(0, GH, band_body, 0)


def _apply_body(x_hbm, luts_hbm, out_hbm, x_v, o_v, lut_v, co0_v, co1_v, wx_v):
    wid = lax.axis_index("s") * NC + lax.axis_index("c")
    lanes = lax.iota(jnp.int32, L)

    def col_body(k, _):
        xpos = (k * L + lanes).astype(jnp.float32)
        tx = (xpos + 0.5) * (1.0 / TW) - 0.5
        x0i = (tx + 1.0).astype(jnp.int32) - 1
        x0 = jnp.clip(x0i, 0, GW - 1)
        x1 = jnp.clip(x0i + 1, 0, GW - 1)
        co0_v[pl.ds(k * L, L)] = x0 * NBINS
        co1_v[pl.ds(k * L, L)] = x1 * NBINS
        wx_v[pl.ds(k * L, L)] = tx - x0i.astype(jnp.float32)
        return 0
    lax.fori_loop(0, W // L, col_body, 0)

    for il in range(IPW):
        img = wid * IPW + il
        bi = img // C
        ci = img % C
        pltpu.sync_copy(luts_hbm.at[img], lut_v)

        def band_body(band, _):
            pltpu.sync_copy(x_hbm.at[bi, ci, pl.ds(band * TH, TH), :], x_v)

            def row_body(r, _):
                y = band * TH + r
                ty = (y.astype(jnp.float32) + 0.5) * (1.0 / TH) - 0.5
                y0i = (ty + 1.0).astype(jnp.int32) - 1
                y0 = jnp.clip(y0i, 0, GH - 1)
                y1 = jnp.clip(y0i + 1, 0, GH - 1)
                wy = ty - y0i.astype(jnp.float32)
                ro0 = y0 * (GW * NBINS)
                ro1 = y1 * (GW * NBINS)

                def ch_body(k, _):
                    xv = x_v[r, pl.ds(k * L, L)]
                    t = xv * (1.0 / 255.0) * 256.0
                    bn = jnp.clip(t.astype(jnp.int32), 0, NBINS - 1)
                    s0 = bn + co0_v[pl.ds(k * L, L)]
                    s1 = bn + co1_v[pl.ds(k * L, L)]
                    wx = wx_v[pl.ds(k * L, L)]
                    v00 = plsc.load_gather(lut_v, [s0 + ro0])
                    v01 = plsc.load_gather(lut_v, [s1 + ro0])
                    v10 = plsc.load_gather(lut_v, [s0 + ro1])
                    v11 = plsc.load_gather(lut_v, [s1 + ro1])
                    omx = 1.0 - wx
                    interp = ((1.0 - wy) * (omx * v00 + wx * v01)
                              + wy * (omx * v10 + wx * v11))
                    o_v[r, pl.ds(k * L, L)] = interp * (1.0 / 255.0) * 255.0
                    return 0
                lax.fori_loop(0, W // L, ch_body, 0)
                return 0

            lax.fori_loop(0, TH, row_body, 0)
            pltpu.sync_copy(o_v, out_hbm.at[bi, ci, pl.ds(band * TH, TH), :])
            return 0

        lax.fori_loop(0, GH, band_body, 0)


_calls = []


def _build_calls():
    mesh = plsc.VectorSubcoreMesh(core_axis_name="c", subcore_axis_name="s",
                                  num_cores=NC, num_subcores=NS)
    params = pltpu.CompilerParams(needs_layout_passes=False)
    hist_call = pl.kernel(
        _hist_body,
        out_type=jax.ShapeDtypeStruct((NIMG, LUT_PER_IMG), jnp.float32),
        mesh=mesh,
        compiler_params=params,
        scratch_types=[
            pltpu.VMEM((TH, W), jnp.float32),
            pltpu.VMEM((NBINS * L,), jnp.float32),
            pltpu.VMEM((NBINS,), jnp.float32),
            pltpu.VMEM((GW * NBINS,), jnp.float32),
        ],
    )
    apply_call = pl.kernel(
        _apply_body,
        out_type=jax.ShapeDtypeStruct((B, C, H, W), jnp.float32),
        mesh=mesh,
        compiler_params=params,
        scratch_types=[
            pltpu.VMEM((TH, W), jnp.float32),
            pltpu.VMEM((TH, W), jnp.float32),
            pltpu.VMEM((LUT_PER_IMG,), jnp.float32),
            pltpu.VMEM((W,), jnp.int32),
            pltpu.VMEM((W,), jnp.int32),
            pltpu.VMEM((W,), jnp.float32),
        ],
    )
    _calls.append((hist_call, apply_call))


@jax.jit
def kernel(x):
    if not _calls:
        _build_calls()
    hist_call, apply_call = _calls[0]
    luts = hist_call(x)
    return apply_call(x, luts)


# speedup vs baseline: 1530.2101x; 1.3619x over previous
import jax
import jax.numpy as jnp
from jax import lax
from jax.experimental import pallas as pl
from jax.experimental.pallas import tpu as pltpu
from jax.experimental.pallas import tpu_sc as plsc

NC, NS, L = 2, 16, 16
NW = NC * NS
B, C, H, W = 32, 3, 512, 512
GH, GW = 8, 8
TH, TW = H // GH, W // GW
NBINS = 256
NPIX = TH * TW
NIMG = B * C
IPW = NIMG // NW
CLIP_VAL = 64.0
HB = 32
NHB = H // HB
def _unpack_bins(w):
    return (w & 255,
            lax.shift_right_logical(w, 8) & 255,
            lax.shift_right_logical(w, 16) & 255,
            lax.shift_right_logical(w, 24))


def _clahe_body(x_hbm, out_hbm, xh_v, ob_v, bins_v, sub_v, hist_v, lut_v,
                co0_v, co1_v, wx_v):
    wid = lax.axis_index("s") * NC + lax.axis_index("c")
    lanes = lax.iota(jnp.int32, L)
    laneoff = lanes * NBINS
    ones = jnp.ones((L,), jnp.float32)
    zeros = jnp.zeros((L,), jnp.float32)

    def col_body(k, _):
        xpos = (k * L + lanes).astype(jnp.float32)
        tx = (xpos + 0.5) * (1.0 / TW) - 0.5
        x0i = (tx + 1.0).astype(jnp.int32) - 1
        x0 = jnp.clip(x0i, 0, GW - 1)
        x1 = jnp.clip(x0i + 1, 0, GW - 1)
        co0_v[pl.ds(k * L, L)] = x0 * NBINS
        co1_v[pl.ds(k * L, L)] = x1 * NBINS
        wx_v[pl.ds(k * L, L)] = tx - x0i.astype(jnp.float32)
        return 0
    lax.fori_loop(0, W // L, col_body, 0)

    def z_body(i, _):
        sub_v[pl.ds(i * L, L)] = zeros
        return 0
    lax.fori_loop(0, (NBINS * L) // L, z_body, 0)

    for il in range(IPW):
        img = wid * IPW + il
        bi = img // C
        ci = img % C

        def hb_in_body(hb, _):
            pltpu.sync_copy(x_hbm.at[bi, ci, pl.ds(hb * HB, HB), :], xh_v)

            def r_body(r, _):
                def g_body(g, _):
                    base = g * 64
                    bns = []
                    for j in range(4):
                        xv = xh_v[r, pl.ds(base + j * L, L)]
                        t = xv * (1.0 / 255.0) * 256.0
                        bns.append(jnp.clip(t.astype(jnp.int32), 0, NBINS - 1))
                    p = (bns[0]
                         | lax.shift_left(bns[1], 8)
                         | lax.shift_left(bns[2], 16)
                         | lax.shift_left(bns[3], 24))
                    bins_v[pl.ds((hb * HB + r) * (W // 4) + g * L, L)] = p
                    return 0
                lax.fori_loop(0, W // 64, g_body, 0)
                return 0
            lax.fori_loop(0, HB, r_body, 0)
            return 0
        lax.fori_loop(0, NHB, hb_in_body, 0)

        def band_body(band, _):
            def tile_body(tc, _):
                def sc_body(r, _):
                    off = (band * TH + r) * (W // 4) + tc * (TW // 4)
                    for bn in _unpack_bins(bins_v[pl.ds(off, L)]):
                        plsc.addupdate_scatter(sub_v, [bn + laneoff], ones)
                    return 0
                lax.fori_loop(0, TH, sc_body, 0)

                def red_body(cc, sacc):
                    acc = sub_v[pl.ds(cc * L, L)]
                    sub_v[pl.ds(cc * L, L)] = zeros
                    for l in range(1, L):
                        acc = acc + sub_v[pl.ds(l * NBINS + cc * L, L)]
                        sub_v[pl.ds(l * NBINS + cc * L, L)] = zeros
                    cl = jnp.minimum(acc, CLIP_VAL)
                    hist_v[pl.ds(cc * L, L)] = cl
                    return sacc + cl
                sacc = lax.fori_loop(0, NBINS // L, red_body, zeros)
                exc = (jnp.float32(NPIX) - jnp.sum(sacc)) * (1.0 / NBINS)

                def lut_body(cc, carry):
                    h = hist_v[pl.ds(cc * L, L)] + exc
                    cs = plsc.cumsum(h) + carry
                    lut = (cs * 255.0 * (1.0 / NPIX)).astype(jnp.int32)
                    lutf = jnp.clip(lut.astype(jnp.float32), 0.0, 255.0)
                    lut_v[pl.ds((band * GW + tc) * NBINS + cc * L, L)] = lutf
                    return carry + jnp.sum(h)
                lax.fori_loop(0, NBINS // L, lut_body, jnp.float32(0.0))
                return 0
            lax.fori_loop(0, GW, tile_body, 0)
            return 0
        lax.fori_loop(0, GH, band_body, 0)

        def hb_out_body(hb, _):
            def row_body(r, _):
                y = hb * HB + r
                ty = (y.astype(jnp.float32) + 0.5) * (1.0 / TH) - 0.5
                y0i = (ty + 1.0).astype(jnp.int32) - 1
                y0 = jnp.clip(y0i, 0, GH - 1)
                y1 = jnp.clip(y0i + 1, 0, GH - 1)
                wy = ty - y0i.astype(jnp.float32)
                ro0 = y0 * (GW * NBINS)
                ro1 = y1 * (GW * NBINS)

                def g_body(g, _):
                    base = g * 64
                    bns = _unpack_bins(bins_v[pl.ds(y * (W // 4) + g * L, L)])
                    for j in range(4):
                        col = base + j * L
                        bn = bns[j]
                        s0 = bn + co0_v[pl.ds(col, L)]
                        s1 = bn + co1_v[pl.ds(col, L)]
                        wx = wx_v[pl.ds(col, L)]
                        v00 = plsc.load_gather(lut_v, [s0 + ro0])
                        v01 = plsc.load_gather(lut_v, [s1 + ro0])
                        v10 = plsc.load_gather(lut_v, [s0 + ro1])
                        v11 = plsc.load_gather(lut_v, [s1 + ro1])
                        omx = 1.0 - wx
                        interp = ((1.0 - wy) * (omx * v00 + wx * v01)
                                  + wy * (omx * v10 + wx * v11))
                        ob_v[r, pl.ds(col, L)] = interp * (1.0 / 255.0) * 255.0
                    return 0
                lax.fori_loop(0, W // 64, g_body, 0)
                return 0
            lax.fori_loop(0, HB, row_body, 0)
            pltpu.sync_copy(ob_v, out_hbm.at[bi, ci, pl.ds(hb * HB, HB), :])
            return 0
        lax.fori_loop(0, NHB, hb_out_body, 0)


_calls = []


def _build_calls():
    mesh = plsc.VectorSubcoreMesh(core_axis_name="c", subcore_axis_name="s",
                                  num_cores=NC, num_subcores=NS)
    params = pltpu.CompilerParams(needs_layout_passes=False)
    clahe_call = pl.kernel(
        _clahe_body,
        out_type=jax.ShapeDtypeStruct((B, C, H, W), jnp.float32),
        mesh=mesh,
        compiler_params=params,
        scratch_types=[
            pltpu.VMEM((HB, W), jnp.float32),
            pltpu.VMEM((HB, W), jnp.float32),
            pltpu.VMEM((H * W // 4,), jnp.int32),
            pltpu.VMEM((NBINS * L,), jnp.float32),
            pltpu.VMEM((NBINS,), jnp.float32),
            pltpu.VMEM((GH * GW * NBINS,), jnp.float32),
            pltpu.VMEM((W,), jnp.int32),
            pltpu.VMEM((W,), jnp.int32),
            pltpu.VMEM((W,), jnp.float32),
        ],
    )
    _calls.append(clahe_call)


@jax.jit
def kernel(x):
    if not _calls:
        _build_calls()
    return _calls[0](x)


# speedup vs baseline: 3866.3220x; 2.5267x over previous
import jax
import jax.numpy as jnp
from jax import lax
from jax.experimental import pallas as pl
from jax.experimental.pallas import tpu as pltpu
from jax.experimental.pallas import tpu_sc as plsc

NC, NS, L = 2, 16, 16
NW = NC * NS
B, C, H, W = 32, 3, 512, 512
GH, GW = 8, 8
TH, TW = H // GH, W // GW
NBINS = 256
NPIX = TH * TW
NIMG = B * C
IPW = NIMG // NW
CLIP_VAL = 64.0
HB = 32
NHB = H // HB
def _unpack_bins(w):
    return (w & 255,
            lax.shift_right_logical(w, 8) & 255,
            lax.shift_right_logical(w, 16) & 255,
            lax.shift_right_logical(w, 24))


def _clahe_body(x_hbm, out_hbm, xh_v, ob_v, bins_v, sub_v, hist_v, lut_v,
                co0_v, co1_v, wx_v):
    wid = lax.axis_index("s") * NC + lax.axis_index("c")
    lanes = lax.iota(jnp.int32, L)
    laneoff = lanes * NBINS
    ones = jnp.ones((L,), jnp.float32)
    zeros = jnp.zeros((L,), jnp.float32)

    def col_body(k, _):
        xpos = (k * L + lanes).astype(jnp.float32)
        tx = (xpos + 0.5) * (1.0 / TW) - 0.5
        x0i = (tx + 1.0).astype(jnp.int32) - 1
        x0 = jnp.clip(x0i, 0, GW - 1)
        x1 = jnp.clip(x0i + 1, 0, GW - 1)
        co0_v[pl.ds(k * L, L)] = x0 * NBINS
        co1_v[pl.ds(k * L, L)] = x1 * NBINS
        wx_v[pl.ds(k * L, L)] = tx - x0i.astype(jnp.float32)
        return 0
    lax.fori_loop(0, W // L, col_body, 0)

    def z_body(i, _):
        sub_v[pl.ds(i * L, L)] = zeros
        return 0
    lax.fori_loop(0, (NBINS * L) // L, z_body, 0)

    for il in range(IPW):
        img = wid * IPW + il
        bi = img // C
        ci = img % C

        def hb_in_body(hb, _):
            pltpu.sync_copy(x_hbm.at[bi, ci, pl.ds(hb * HB, HB), :], xh_v)

            @plsc.parallel_loop(0, HB * (W // 64), unroll=2)
            def _(i):
                r = i // (W // 64)
                g = i % (W // 64)
                base = g * 64
                bns = []
                for j in range(4):
                    xv = xh_v[r, pl.ds(base + j * L, L)]
                    t = xv * (1.0 / 255.0) * 256.0
                    bns.append(jnp.clip(t.astype(jnp.int32), 0, NBINS - 1))
                p = (bns[0]
                     | lax.shift_left(bns[1], 8)
                     | lax.shift_left(bns[2], 16)
                     | lax.shift_left(bns[3], 24))
                bins_v[pl.ds((hb * HB + r) * (W // 4) + g * L, L)] = p
            return 0
        lax.fori_loop(0, NHB, hb_in_body, 0)

        def band_body(band, _):
            def tile_body(tc, _):
                @plsc.parallel_loop(0, TH, unroll=2)
                def _(r):
                    off = (band * TH + r) * (W // 4) + tc * (TW // 4)
                    for bn in _unpack_bins(bins_v[pl.ds(off, L)]):
                        plsc.addupdate_scatter(sub_v, [bn + laneoff], ones)

                def red_body(cc, sacc):
                    acc = sub_v[pl.ds(cc * L, L)]
                    sub_v[pl.ds(cc * L, L)] = zeros
                    for l in range(1, L):
                        acc = acc + sub_v[pl.ds(l * NBINS + cc * L, L)]
                        sub_v[pl.ds(l * NBINS + cc * L, L)] = zeros
                    cl = jnp.minimum(acc, CLIP_VAL)
                    hist_v[pl.ds(cc * L, L)] = cl
                    return sacc + cl
                sacc = plsc.parallel_loop(
                    0, NBINS // L, carry=zeros)(
                        lambda cc, sacc: red_body(cc, sacc))
                exc = (jnp.float32(NPIX) - jnp.sum(sacc)) * (1.0 / NBINS)

                def lut_body(cc, carry):
                    h = hist_v[pl.ds(cc * L, L)] + exc
                    cs = plsc.cumsum(h) + carry
                    lut = (cs * 255.0 * (1.0 / NPIX)).astype(jnp.int32)
                    lutf = jnp.clip(lut.astype(jnp.float32), 0.0, 255.0)
                    lut_v[pl.ds((band * GW + tc) * NBINS + cc * L, L)] = lutf
                    return carry + jnp.sum(h)
                plsc.parallel_loop(
                    0, NBINS // L, carry=jnp.float32(0.0))(
                        lambda cc, carry: lut_body(cc, carry))
                return 0
            lax.fori_loop(0, GW, tile_body, 0)
            return 0
        lax.fori_loop(0, GH, band_body, 0)

        def hb_out_body(hb, _):
            def row_body(r, _):
                y = hb * HB + r
                ty = (y.astype(jnp.float32) + 0.5) * (1.0 / TH) - 0.5
                y0i = (ty + 1.0).astype(jnp.int32) - 1
                y0 = jnp.clip(y0i, 0, GH - 1)
                y1 = jnp.clip(y0i + 1, 0, GH - 1)
                wy = ty - y0i.astype(jnp.float32)
                ro0 = y0 * (GW * NBINS)
                ro1 = y1 * (GW * NBINS)

                @plsc.parallel_loop(0, W // 64, unroll=2)
                def _(g):
                    base = g * 64
                    bns = _unpack_bins(bins_v[pl.ds(y * (W // 4) + g * L, L)])
                    for j in range(4):
                        col = base + j * L
                        bn = bns[j]
                        s0 = bn + co0_v[pl.ds(col, L)]
                        s1 = bn + co1_v[pl.ds(col, L)]
                        wx = wx_v[pl.ds(col, L)]
                        v00 = plsc.load_gather(lut_v, [s0 + ro0])
                        v01 = plsc.load_gather(lut_v, [s1 + ro0])
                        v10 = plsc.load_gather(lut_v, [s0 + ro1])
                        v11 = plsc.load_gather(lut_v, [s1 + ro1])
                        omx = 1.0 - wx
                        interp = ((1.0 - wy) * (omx * v00 + wx * v01)
                                  + wy * (omx * v10 + wx * v11))
                        ob_v[r, pl.ds(col, L)] = interp * (1.0 / 255.0) * 255.0
                return 0
            lax.fori_loop(0, HB, row_body, 0)
            pltpu.sync_copy(ob_v, out_hbm.at[bi, ci, pl.ds(hb * HB, HB), :])
            return 0
        lax.fori_loop(0, NHB, hb_out_body, 0)


_calls = []


def _build_calls():
    mesh = plsc.VectorSubcoreMesh(core_axis_name="c", subcore_axis_name="s",
                                  num_cores=NC, num_subcores=NS)
    params = pltpu.CompilerParams(needs_layout_passes=False)
    clahe_call = pl.kernel(
        _clahe_body,
        out_type=jax.ShapeDtypeStruct((B, C, H, W), jnp.float32),
        mesh=mesh,
        compiler_params=params,
        scratch_types=[
            pltpu.VMEM((HB, W), jnp.float32),
            pltpu.VMEM((HB, W), jnp.float32),
            pltpu.VMEM((H * W // 4,), jnp.int32),
            pltpu.VMEM((NBINS * L,), jnp.float32),
            pltpu.VMEM((NBINS,), jnp.float32),
            pltpu.VMEM((GH * GW * NBINS,), jnp.float32),
            pltpu.VMEM((W,), jnp.int32),
            pltpu.VMEM((W,), jnp.int32),
            pltpu.VMEM((W,), jnp.float32),
        ],
    )
    _calls.append(clahe_call)


@jax.jit
def kernel(x):
    if not _calls:
        _build_calls()
    return _calls[0](x)
